# Initial kernel scaffold; baseline (speedup 1.0000x reference)
import jax
import jax.numpy as jnp
from jax.experimental import pallas as pl


def kernel(x, edge_index, W_l, b_l, W_r, ln_gamma, ln_beta):
    raise NotImplementedError("write your pallas kernel here")


# speedup vs baseline: 3.6474x; 3.6474x over previous
import functools

import jax
import jax.numpy as jnp
from jax import lax
from jax.experimental import pallas as pl
from jax.experimental.pallas import tpu as pltpu
from jax.experimental.pallas import tpu_sc as plsc

_NT = 16
_K = 80


def _sc_aggregate(xa, xb, src, dst, zrow, ones_rows):
    n, h = xa.shape
    e = src.shape[0]
    ept = e // _NT
    nch = ept // _K
    wch = n // _K
    wloop = (wch + _NT - 1) // _NT
    cpc = e // 2 // _K
    cloop = (cpc + _NT - 1) // _NT

    mesh = plsc.VectorSubcoreMesh(core_axis_name="c", subcore_axis_name="s")

    @functools.partial(
        pl.kernel,
        mesh=mesh,
        out_type=(
            jax.ShapeDtypeStruct((n, h), jnp.float32),
            jax.ShapeDtypeStruct((n, h), jnp.float32),
            jax.ShapeDtypeStruct((n, h), jnp.float32),
            jax.ShapeDtypeStruct((n, h), jnp.float32),
        ),
        scratch_types=(
            pltpu.VMEM((_K,), jnp.int32),
            pltpu.VMEM((_K,), jnp.int32),
            pltpu.VMEM((_K, h), jnp.float32),
            pltpu.VMEM_SHARED((n, h), jnp.float32),
            pltpu.SemaphoreType.DMA,
        ),
    )
    def body(xa_h, xb_h, src_h, dst_h, zrow_h, ones_h,
             out_a, out_b, out_c0, out_c1,
             idx_s, idx_d, rows_v, acc, sem):
        c = lax.axis_index("c")
        s = lax.axis_index("s")

        pltpu.sync_copy(zrow_h, rows_v)
        for k in range(wloop):
            cid = k * _NT + s

            @pl.when(cid < wch)
            def _():
                r0 = pl.multiple_of(cid * _K, _K)
                pltpu.sync_copy(rows_v, acc.at[pl.ds(r0, _K)])

        plsc.subcore_barrier()

        def chunk(g, carry):
            off = pl.multiple_of(s * ept + g * _K, _K)
            pltpu.sync_copy(src_h.at[pl.ds(off, _K)], idx_s)
            pltpu.sync_copy(dst_h.at[pl.ds(off, _K)], idx_d)

            @pl.when(c == 0)
            def _():
                pltpu.async_copy(xa_h.at[idx_s], rows_v, sem).wait()

            @pl.when(c == 1)
            def _():
                pltpu.async_copy(xb_h.at[idx_s], rows_v, sem).wait()

            pltpu.sync_copy(rows_v, acc.at[idx_d], add=True)
            return carry

        lax.fori_loop(0, nch, chunk, 0)
        plsc.subcore_barrier()

        for k in range(wloop):
            cid = k * _NT + s

            @pl.when(cid < wch)
            def _():
                r0 = pl.multiple_of(cid * _K, _K)
                pltpu.sync_copy(acc.at[pl.ds(r0, _K)], rows_v)

                @pl.when(c == 0)
                def _():
                    pltpu.sync_copy(rows_v, out_a.at[pl.ds(r0, _K)])

                @pl.when(c == 1)
                def _():
                    pltpu.sync_copy(rows_v, out_b.at[pl.ds(r0, _K)])

        plsc.subcore_barrier()

        pltpu.sync_copy(zrow_h, rows_v)
        for k in range(wloop):
            cid = k * _NT + s

            @pl.when(cid < wch)
            def _():
                r0 = pl.multiple_of(cid * _K, _K)
                pltpu.sync_copy(rows_v, acc.at[pl.ds(r0, _K)])

        plsc.subcore_barrier()
        pltpu.sync_copy(ones_h, rows_v)

        def cchunk(k, carry):
            j = k * _NT + s

            @pl.when(j < cpc)
            def _():
                off = pl.multiple_of((c * cpc + j) * _K, _K)
                pltpu.sync_copy(dst_h.at[pl.ds(off, _K)], idx_d)
                pltpu.sync_copy(rows_v, acc.at[idx_d], add=True)

            return carry

        lax.fori_loop(0, cloop, cchunk, 0)
        plsc.subcore_barrier()

        for k in range(wloop):
            cid = k * _NT + s

            @pl.when(cid < wch)
            def _():
                r0 = pl.multiple_of(cid * _K, _K)
                pltpu.sync_copy(acc.at[pl.ds(r0, _K)], rows_v)

                @pl.when(c == 0)
                def _():
                    pltpu.sync_copy(rows_v, out_c0.at[pl.ds(r0, _K)])

                @pl.when(c == 1)
                def _():
                    pltpu.sync_copy(rows_v, out_c1.at[pl.ds(r0, _K)])

    return body(xa, xb, src, dst, zrow, ones_rows)


def _tc_finish(agg_a, agg_b, cnt0, cnt1, x, W_l, b_l, W_r, gamma, beta):
    n, d = x.shape
    h = d // 2
    r = 1000
    grid = n // r

    def body(a_ref, b_ref, c0_ref, c1_ref, x_ref, wl_ref, bl_ref, wr_ref,
             g_ref, be_ref, o_ref):
        agg = jnp.concatenate([a_ref[...], b_ref[...]], axis=1)
        count = c0_ref[...][:, 0:1] + c1_ref[...][:, 0:1]
        count = jnp.maximum(count, 1.0)
        agg = agg / count
        out = lax.dot_general(agg, wl_ref[...], (((1,), (1,)), ((), ())),
                              preferred_element_type=jnp.float32)
        out = out + lax.dot_general(x_ref[...], wr_ref[...],
                                    (((1,), (1,)), ((), ())),
                                    preferred_element_type=jnp.float32)
        out = out + bl_ref[...]
        mu = jnp.mean(out, axis=-1, keepdims=True)
        dev = out - mu
        var = jnp.mean(dev * dev, axis=-1, keepdims=True)
        out = dev * lax.rsqrt(var + 1e-5) * g_ref[...] + be_ref[...]
        o_ref[...] = jnp.maximum(out, 0.0)

    return pl.pallas_call(
        body,
        grid=(grid,),
        in_specs=[
            pl.BlockSpec((r, h), lambda i: (i, 0)),
            pl.BlockSpec((r, h), lambda i: (i, 0)),
            pl.BlockSpec((r, h), lambda i: (i, 0)),
            pl.BlockSpec((r, h), lambda i: (i, 0)),
            pl.BlockSpec((r, d), lambda i: (i, 0)),
            pl.BlockSpec((d, d), lambda i: (0, 0)),
            pl.BlockSpec((1, d), lambda i: (0, 0)),
            pl.BlockSpec((d, d), lambda i: (0, 0)),
            pl.BlockSpec((1, d), lambda i: (0, 0)),
            pl.BlockSpec((1, d), lambda i: (0, 0)),
        ],
        out_specs=pl.BlockSpec((r, d), lambda i: (i, 0)),
        out_shape=jax.ShapeDtypeStruct((n, d), jnp.float32),
    )(agg_a, agg_b, cnt0, cnt1, x, W_l, b_l.reshape(1, d), W_r,
      gamma.reshape(1, d), beta.reshape(1, d))


def kernel(x, edge_index, W_l, b_l, W_r, ln_gamma, ln_beta):
    n, d = x.shape
    h = d // 2
    src = edge_index[0]
    dst = edge_index[1]
    xa = x[:, :h]
    xb = x[:, h:]
    zrow = jnp.zeros((_K, h), jnp.float32)
    ones_rows = jnp.zeros((_K, h), jnp.float32).at[:, 0].set(1.0)
    agg_a, agg_b, cnt0, cnt1 = _sc_aggregate(xa, xb, src, dst, zrow,
                                             ones_rows)
    return _tc_finish(agg_a, agg_b, cnt0, cnt1, x, W_l, b_l, W_r,
                      ln_gamma, ln_beta)


# speedup vs baseline: 4.6558x; 1.2765x over previous
import functools

import jax
import jax.numpy as jnp
from jax import lax
from jax.experimental import pallas as pl
from jax.experimental.pallas import tpu as pltpu
from jax.experimental.pallas import tpu_sc as plsc

_NT = 16
_K = 128
_WK = 80


def _sc_aggregate(xa, xb, src, dst, zrow, ones_rows):
    n, h = xa.shape
    e = src.shape[0]
    npair = e // (2 * _K)
    ploop = (npair + _NT - 1) // _NT
    cloop = (npair + 2 * _NT - 1) // (2 * _NT)
    wch = n // _WK
    wloop = (wch + _NT - 1) // _NT

    mesh = plsc.VectorSubcoreMesh(core_axis_name="c", subcore_axis_name="s")

    @functools.partial(
        pl.kernel,
        mesh=mesh,
        out_type=(
            jax.ShapeDtypeStruct((n, h), jnp.float32),
            jax.ShapeDtypeStruct((n, h), jnp.float32),
            jax.ShapeDtypeStruct((n, h), jnp.float32),
            jax.ShapeDtypeStruct((n, h), jnp.float32),
        ),
        scratch_types=(
            pltpu.VMEM((_K,), jnp.int32),
            pltpu.VMEM((_K,), jnp.int32),
            pltpu.VMEM((_K,), jnp.int32),
            pltpu.VMEM((_K,), jnp.int32),
            pltpu.VMEM((_K, h), jnp.float32),
            pltpu.VMEM((_K, h), jnp.float32),
            pltpu.VMEM_SHARED((n, h), jnp.float32),
            pltpu.SemaphoreType.DMA,
            pltpu.SemaphoreType.DMA,
            pltpu.SemaphoreType.DMA,
            pltpu.SemaphoreType.DMA,
        ),
    )
    def body(xa_h, xb_h, src_h, dst_h, zrow_h, ones_h,
             out_a, out_b, out_c0, out_c1,
             idx_s0, idx_s1, idx_d0, idx_d1, rows0, rows1, acc,
             sga, sgb, ssa, ssb):
        c = lax.axis_index("c")
        s = lax.axis_index("s")

        def zero_acc():
            pltpu.sync_copy(zrow_h, rows0)
            for k in range(wloop):
                cid = k * _NT + s

                @pl.when(cid < wch)
                def _():
                    r0 = pl.multiple_of(cid * _WK, _WK)
                    pltpu.sync_copy(rows0.at[pl.ds(0, _WK)],
                                    acc.at[pl.ds(r0, _WK)])

        def write_acc(dst0_h, dst1_h):
            for k in range(wloop):
                cid = k * _NT + s

                @pl.when(cid < wch)
                def _():
                    r0 = pl.multiple_of(cid * _WK, _WK)
                    pltpu.sync_copy(acc.at[pl.ds(r0, _WK)],
                                    rows0.at[pl.ds(0, _WK)])

                    @pl.when(c == 0)
                    def _():
                        pltpu.sync_copy(rows0.at[pl.ds(0, _WK)],
                                        dst0_h.at[pl.ds(r0, _WK)])

                    @pl.when(c == 1)
                    def _():
                        pltpu.sync_copy(rows0.at[pl.ds(0, _WK)],
                                        dst1_h.at[pl.ds(r0, _WK)])

        zero_acc()
        plsc.subcore_barrier()

        def chunk(k, carry):
            p = k * _NT + s

            @pl.when(p < npair)
            def _():
                offa = pl.multiple_of((2 * p) * _K, _K)
                offb = pl.multiple_of((2 * p + 1) * _K, _K)
                pltpu.sync_copy(src_h.at[pl.ds(offa, _K)], idx_s0)
                pltpu.sync_copy(src_h.at[pl.ds(offb, _K)], idx_s1)
                pltpu.sync_copy(dst_h.at[pl.ds(offa, _K)], idx_d0)
                pltpu.sync_copy(dst_h.at[pl.ds(offb, _K)], idx_d1)

                @pl.when(c == 0)
                def _():
                    ga = pltpu.async_copy(xa_h.at[idx_s0], rows0, sga)
                    gb = pltpu.async_copy(xa_h.at[idx_s1], rows1, sgb)
                    ga.wait()
                    sa = pltpu.async_copy(rows0, acc.at[idx_d0], ssa,
                                          add=True)
                    gb.wait()
                    sb = pltpu.async_copy(rows1, acc.at[idx_d1], ssb,
                                          add=True)
                    sa.wait()
                    sb.wait()

                @pl.when(c == 1)
                def _():
                    ga = pltpu.async_copy(xb_h.at[idx_s0], rows0, sga)
                    gb = pltpu.async_copy(xb_h.at[idx_s1], rows1, sgb)
                    ga.wait()
                    sa = pltpu.async_copy(rows0, acc.at[idx_d0], ssa,
                                          add=True)
                    gb.wait()
                    sb = pltpu.async_copy(rows1, acc.at[idx_d1], ssb,
                                          add=True)
                    sa.wait()
                    sb.wait()

            return carry

        lax.fori_loop(0, ploop, chunk, 0)
        plsc.subcore_barrier()
        write_acc(out_a, out_b)
        plsc.subcore_barrier()

        zero_acc()
        plsc.subcore_barrier()
        pltpu.sync_copy(ones_h, rows0)

        def cchunk(k, carry):
            p = k * 2 * _NT + c * _NT + s

            @pl.when(p < npair)
            def _():
                offa = pl.multiple_of((2 * p) * _K, _K)
                offb = pl.multiple_of((2 * p + 1) * _K, _K)
                pltpu.sync_copy(dst_h.at[pl.ds(offa, _K)], idx_d0)
                pltpu.sync_copy(dst_h.at[pl.ds(offb, _K)], idx_d1)
                sa = pltpu.async_copy(rows0, acc.at[idx_d0], ssa, add=True)
                sb = pltpu.async_copy(rows0, acc.at[idx_d1], ssb, add=True)
                sa.wait()
                sb.wait()

            return carry

        lax.fori_loop(0, cloop, cchunk, 0)
        plsc.subcore_barrier()
        write_acc(out_c0, out_c1)

    return body(xa, xb, src, dst, zrow, ones_rows)


def _tc_finish(agg_a, agg_b, cnt0, cnt1, x, W_l, b_l, W_r, gamma, beta):
    n, d = x.shape
    h = d // 2
    r = 1000
    grid = n // r

    def body(a_ref, b_ref, c0_ref, c1_ref, x_ref, wl_ref, bl_ref, wr_ref,
             g_ref, be_ref, o_ref):
        agg = jnp.concatenate([a_ref[...], b_ref[...]], axis=1)
        count = c0_ref[...][:, 0:1] + c1_ref[...][:, 0:1]
        count = jnp.maximum(count, 1.0)
        agg = agg / count
        out = lax.dot_general(agg, wl_ref[...], (((1,), (1,)), ((), ())),
                              preferred_element_type=jnp.float32)
        out = out + lax.dot_general(x_ref[...], wr_ref[...],
                                    (((1,), (1,)), ((), ())),
                                    preferred_element_type=jnp.float32)
        out = out + bl_ref[...]
        mu = jnp.mean(out, axis=-1, keepdims=True)
        dev = out - mu
        var = jnp.mean(dev * dev, axis=-1, keepdims=True)
        out = dev * lax.rsqrt(var + 1e-5) * g_ref[...] + be_ref[...]
        o_ref[...] = jnp.maximum(out, 0.0)

    return pl.pallas_call(
        body,
        grid=(grid,),
        in_specs=[
            pl.BlockSpec((r, h), lambda i: (i, 0)),
            pl.BlockSpec((r, h), lambda i: (i, 0)),
            pl.BlockSpec((r, h), lambda i: (i, 0)),
            pl.BlockSpec((r, h), lambda i: (i, 0)),
            pl.BlockSpec((r, d), lambda i: (i, 0)),
            pl.BlockSpec((d, d), lambda i: (0, 0)),
            pl.BlockSpec((1, d), lambda i: (0, 0)),
            pl.BlockSpec((d, d), lambda i: (0, 0)),
            pl.BlockSpec((1, d), lambda i: (0, 0)),
            pl.BlockSpec((1, d), lambda i: (0, 0)),
        ],
        out_specs=pl.BlockSpec((r, d), lambda i: (i, 0)),
        out_shape=jax.ShapeDtypeStruct((n, d), jnp.float32),
    )(agg_a, agg_b, cnt0, cnt1, x, W_l, b_l.reshape(1, d), W_r,
      gamma.reshape(1, d), beta.reshape(1, d))


def kernel(x, edge_index, W_l, b_l, W_r, ln_gamma, ln_beta):
    n, d = x.shape
    h = d // 2
    src = edge_index[0]
    dst = edge_index[1]
    xa = x[:, :h]
    xb = x[:, h:]
    zrow = jnp.zeros((_K, h), jnp.float32)
    ones_rows = jnp.zeros((_K, h), jnp.float32).at[:, 0].set(1.0)
    assert src.shape[0] % (2 * _K) == 0 and n % _WK == 0
    agg_a, agg_b, cnt0, cnt1 = _sc_aggregate(xa, xb, src, dst, zrow,
                                             ones_rows)
    return _tc_finish(agg_a, agg_b, cnt0, cnt1, x, W_l, b_l, W_r,
                      ln_gamma, ln_beta)
